# Initial kernel scaffold; baseline (speedup 1.0000x reference)
import jax
import jax.numpy as jnp
from jax.experimental import pallas as pl


def kernel(base_bxyz, bcenter, e_base, e_voxel):
    raise NotImplementedError("write your pallas kernel here")


# speedup vs baseline: 1.8114x; 1.8114x over previous
import functools

import jax
import jax.numpy as jnp
from jax import lax
from jax.experimental import pallas as pl
from jax.experimental.pallas import tpu as pltpu
from jax.experimental.pallas import tpu_sc as plsc

L = 16
NC = 2
NS = 16
NW = NC * NS
SUB = 80
KSUB = 5
CH = SUB * KSUB

_CTAB = [
    [4, 0, 1, 2, 3, 1, 1, 1, 2, 2, 3, 5, 5, 5, 5, 5],
    [4, 4, 4, 4, 4, 1, 2, 3, 2, 3, 3, 5, 5, 5, 5, 5],
    [1, 1, 1, 2, 2, 2, 3, 3, 3, 5, 5, 5, 5, 5, 5, 5],
    [9, 9, 9, 10, 10, 10, 11, 11, 11, 12, 12, 12, 12, 12, 12, 12],
    [3, 4, 5, 6, 7, 8, 9, 10, 11, 12, 12, 12, 12, 12, 12, 12],
    [6, 7, 8, 9, 10, 11, 12, 12, 12, 12, 12, 12, 12, 12, 12, 12],
    [0, 1, 2, 0, 1, 2, 12, 12, 12, 12, 12, 12, 12, 12, 12, 12],
    [1, 1, 1, -1, -1, -1, 0, 0, 0, 0, 0, 0, 0, 0, 0, 0],
    [0, 0, 1, 2, 3, 4, 5, 6, 7, 8, 9, 10, 11, 12, 13, 14],
    [15] * 16,
    [1] + [0] * 15,
    [1, 2, 3, 4, 5, 6, 7, 8, 9, 10, 11, 12, 13, 14, 15, 15],
    [0] * 15 + [1],
]


def _lane():
    return lax.iota(jnp.int32, L)


def _make_sc_moments(E, V, NITER, Vp, RS):
    mesh = plsc.VectorSubcoreMesh(core_axis_name="c", subcore_axis_name="s")

    @functools.partial(
        pl.kernel,
        out_type=jax.ShapeDtypeStruct((NC, Vp, L), jnp.float32),
        mesh=mesh,
        compiler_params=pltpu.CompilerParams(use_tc_tiling_on_sc=False),
        scratch_types=[
            pltpu.VMEM_SHARED((Vp, L), jnp.float32),
            pltpu.VMEM((len(_CTAB), L), jnp.int32),
            pltpu.VMEM((KSUB, SUB), jnp.int32),
            pltpu.VMEM((KSUB, SUB), jnp.int32),
            pltpu.VMEM((CH, L), jnp.float32),
            pltpu.VMEM((CH, L), jnp.float32),
            pltpu.SemaphoreType.DMA,
        ],
    )
    def sc1(base_ref, eb_ref, ev_ref, ctab_ref, zeros_ref, out_ref,
            acc, ctab, idxb, idxv, rows, pay, sem):
        cid = lax.axis_index("c")
        sid = lax.axis_index("s")
        gw = cid * NS + sid
        r0 = sid * RS
        pltpu.sync_copy(ctab_ref, ctab)
        pltpu.sync_copy(zeros_ref, acc.at[pl.ds(r0, RS)])
        idxa_v = ctab[0]
        idxb_v = ctab[1]
        plsc.subcore_barrier()

        def it(i, carry):
            t = gw * NITER + i
            pltpu.sync_copy(eb_ref.at[t], idxb)
            pltpu.sync_copy(ev_ref.at[t], idxv)
            cps = [
                pltpu.async_copy(base_ref.at[idxb.at[k]],
                                 rows.at[pl.ds(k * SUB, SUB)], sem)
                for k in range(KSUB)
            ]
            for cp in cps:
                cp.wait()

            def grp(g, c):
                for u in range(L):
                    r = g * L + u
                    prow = rows[r]
                    a = jnp.take_along_axis(prow, idxa_v, axis=0)
                    b = jnp.take_along_axis(prow, idxb_v, axis=0)
                    pay[r] = a * b
                return c

            lax.fori_loop(0, CH // L, grp, 0)
            for k in range(KSUB):
                pltpu.sync_copy(pay.at[pl.ds(k * SUB, SUB)],
                                acc.at[idxv.at[k]], add=True)
            return carry

        lax.fori_loop(0, NITER, it, 0)
        plsc.subcore_barrier()
        pltpu.sync_copy(acc.at[pl.ds(r0, RS)], out_ref.at[cid, pl.ds(r0, RS)])

    return sc1


def _make_sc_proj(E, V, Vp, RS):
    mesh = plsc.VectorSubcoreMesh(core_axis_name="c", subcore_axis_name="s")
    TCH = E // CH
    DUMMY = V

    @functools.partial(
        pl.kernel,
        out_type=jax.ShapeDtypeStruct((NC, Vp, L), jnp.float32),
        mesh=mesh,
        compiler_params=pltpu.CompilerParams(use_tc_tiling_on_sc=False),
        scratch_types=[
            pltpu.VMEM_SHARED((Vp, L), jnp.float32),
            pltpu.VMEM((len(_CTAB), L), jnp.int32),
            pltpu.VMEM((L,), jnp.int32),
            pltpu.VMEM((KSUB, SUB), jnp.int32),
            pltpu.VMEM((KSUB, SUB), jnp.int32),
            pltpu.VMEM((KSUB, SUB), jnp.int32),
            pltpu.VMEM((CH, L), jnp.float32),
            pltpu.VMEM((CH, L), jnp.float32),
            pltpu.VMEM((CH, L), jnp.float32),
            pltpu.VMEM((L,), jnp.int32),
            pltpu.VMEM((L,), jnp.float32),
            pltpu.SemaphoreType.DMA,
        ],
    )
    def sc3(base_ref, vox_ref, eb_ref, ev_ref, starts_ref, ctab_ref,
            zeros_ref, out_ref,
            acc, ctab, starts, idxb, idxv, sidx, rows, vrows, pay,
            prevbuf, rbuf, sem):
        cid = lax.axis_index("c")
        sid = lax.axis_index("s")
        gw = cid * NS + sid
        lane = _lane()
        r0 = sid * RS
        pltpu.sync_copy(ctab_ref, ctab)
        pltpu.sync_copy(starts_ref.at[gw], starts)
        pltpu.sync_copy(zeros_ref, acc.at[pl.ds(r0, RS)])
        idxp_v = ctab[2]
        idxm_v = ctab[3]
        sh3_v = ctab[4]
        sh6_v = ctab[5]
        packf_v = ctab[6]
        signf = ctab[7].astype(jnp.float32)
        shd1_v = ctab[8]
        spl15_v = ctab[9]
        ind0_v = ctab[10]
        shu1_v = ctab[11]
        ind15_f = ctab[12].astype(jnp.float32)
        zsplat = jnp.zeros((L,), jnp.int32)
        sev = starts[...]
        sv = sev[0]
        es = sev[1]
        t0 = sv // CH
        prevbuf[...] = jnp.full((L,), -1, jnp.int32)
        rbuf[...] = jnp.zeros((L,), jnp.float32)
        plsc.subcore_barrier()

        def chunk(i, carry):
            t = t0 + i

            @pl.when(t * CH < es)
            def _():
                pltpu.sync_copy(eb_ref.at[t], idxb)
                pltpu.sync_copy(ev_ref.at[t], idxv)
                cps = []
                for k in range(KSUB):
                    cps.append(pltpu.async_copy(
                        base_ref.at[idxb.at[k]],
                        rows.at[pl.ds(k * SUB, SUB)], sem))
                    cps.append(pltpu.async_copy(
                        vox_ref.at[idxv.at[k]],
                        vrows.at[pl.ds(k * SUB, SUB)], sem))
                for cp in cps:
                    cp.wait()

                def fin(gp, vvp, okp, samep, nextsame, mgf):
                    shifted = jnp.take_along_axis(samep, shu1_v, axis=0)
                    nxt = shifted * (1.0 - ind15_f) + nextsame * ind15_f
                    ki = ((1.0 - nxt) * okp.astype(jnp.float32) * mgf
                          ).astype(jnp.int32)
                    kkp = gp // KSUB
                    jjp = (gp % KSUB) * L
                    sidx[kkp, pl.ds(jjp, L)] = (vvp * ki
                                                + DUMMY * (1 - ki))

                def grp(g, st):
                    prevv, R, samep, vvp, okp = st
                    kk = g // KSUB
                    jj = (g % KSUB) * L
                    vv = idxv[kk, pl.ds(jj, L)]
                    sh = jnp.take_along_axis(vv, shd1_v, axis=0)
                    sh0 = prevv * ind0_v + sh * (1 - ind0_v)
                    same_f = (1 - jnp.minimum(jnp.abs(vv - sh0), 1)
                              ).astype(jnp.float32)
                    gi = jnp.full((L,), t * CH + g * L, jnp.int32) + lane
                    okv = (jnp.minimum(jnp.maximum(gi - sv + 1, 0), 1)
                           * jnp.minimum(jnp.maximum(es - gi, 0), 1))
                    firstg = jnp.take_along_axis(same_f, zsplat, axis=0)
                    mgf = jnp.minimum(g, 1).astype(jnp.float32)
                    fin(jnp.maximum(g - 1, 0), vvp, okp, samep, firstg, mgf)
                    for u in range(L):
                        r = g * L + u
                        prow = rows[r]
                        vrow = vrows[r]
                        dv = (jnp.take_along_axis(prow, idxp_v, axis=0)
                              - jnp.take_along_axis(vrow, idxm_v, axis=0))
                        q = dv * vrow
                        ss = (q + jnp.take_along_axis(q, sh3_v, axis=0)
                              + jnp.take_along_axis(q, sh6_v, axis=0))
                        f = jnp.take_along_axis(ss, packf_v, axis=0) * signf
                        su = jnp.take_along_axis(
                            same_f, jnp.full((L,), u, jnp.int32), axis=0)
                        R = su * jnp.maximum(R, f) + (1.0 - su) * f
                        pay[r] = R * signf
                    prev2 = jnp.take_along_axis(vv, spl15_v, axis=0)
                    return (prev2, R, same_f, vv, okv)

                st = lax.fori_loop(
                    0, CH // L, grp,
                    (prevbuf[...], rbuf[...], jnp.zeros((L,), jnp.float32),
                     jnp.full((L,), -1, jnp.int32), jnp.zeros((L,), jnp.int32)))
                prevbuf[...] = st[0]
                rbuf[...] = st[1]
                fin(CH // L - 1, st[3], st[4], st[2],
                    jnp.zeros((L,), jnp.float32), jnp.float32(1.0))
                for k in range(KSUB):
                    pltpu.sync_copy(pay.at[pl.ds(k * SUB, SUB)],
                                    acc.at[sidx.at[k]])

            return carry

        lax.fori_loop(0, TCH, chunk, 0)
        plsc.subcore_barrier()
        pltpu.sync_copy(acc.at[pl.ds(r0, RS)], out_ref.at[cid, pl.ds(r0, RS)])

    return sc3


def _tc_starts(ev2):
    Ew = ev2.shape[1]
    E = NW * Ew

    def body(ev_ref, out_ref):
        v2 = ev_ref[...]
        head = jnp.concatenate([v2[:1, :1] - 1, v2[:-1, -1:]], axis=0)
        prev2 = jnp.concatenate([head, v2[:, :-1]], axis=1)
        bound = v2 != prev2
        gidx = (lax.broadcasted_iota(jnp.int32, (NW, Ew), 0) * Ew
                + lax.broadcasted_iota(jnp.int32, (NW, Ew), 1))
        idxs = jnp.where(bound, gidx, E)
        bmin = jnp.min(idxs, axis=1)
        i32 = lax.iota(jnp.int32, NW)
        suf = jnp.min(
            jnp.where(i32[None, :] >= i32[:, None], bmin[None, :], E),
            axis=1)
        out_ref[...] = suf

    return pl.pallas_call(
        body,
        in_specs=[pl.BlockSpec(memory_space=pltpu.VMEM)],
        out_specs=pl.BlockSpec(memory_space=pltpu.VMEM),
        out_shape=jax.ShapeDtypeStruct((NW,), jnp.int32),
    )(ev2)


def _tc_stats(parts, bcenter):
    V = bcenter.shape[0]
    BR = 2048
    grid = (pl.cdiv(V, BR),)

    def body(p_ref, c_ref, vol_ref, bxyz_ref, cov_ref):
        p = p_ref[0] + p_ref[1]
        n = p[:, 0]
        safe = jnp.maximum(n, 1.0)
        mean4 = p[:, 1:5] / safe[:, None]
        mask = n > 0.5
        bxyz_ref[...] = jnp.where(mask[:, None], mean4, c_ref[...])
        vol_ref[...] = n
        mx, my, mz = mean4[:, 1], mean4[:, 2], mean4[:, 3]
        s2 = p[:, 5:11] / safe[:, None]
        prods = jnp.stack(
            [mx * mx, mx * my, mx * mz, my * my, my * mz, mz * mz], axis=1)
        cov_ref[...] = s2 - prods

    return pl.pallas_call(
        body,
        grid=grid,
        in_specs=[
            pl.BlockSpec((2, BR, L), lambda i: (0, i, 0)),
            pl.BlockSpec((BR, 4), lambda i: (i, 0)),
        ],
        out_specs=[
            pl.BlockSpec((BR,), lambda i: (i,)),
            pl.BlockSpec((BR, 4), lambda i: (i, 0)),
            pl.BlockSpec((BR, 6), lambda i: (i, 0)),
        ],
        out_shape=[
            jax.ShapeDtypeStruct((V,), jnp.float32),
            jax.ShapeDtypeStruct((V, 4), jnp.float32),
            jax.ShapeDtypeStruct((V, 6), jnp.float32),
        ],
    )(parts, bcenter)


def _tc_combine(parts):
    Vp = parts.shape[1]
    BR = 2048
    grid = (pl.cdiv(Vp, BR),)

    def body(p_ref, mx_ref, mn_ref):
        p = p_ref[0] + p_ref[1]
        mx_ref[...] = p[:, 0:3]
        mn_ref[...] = p[:, 3:6]

    return pl.pallas_call(
        body,
        grid=grid,
        in_specs=[pl.BlockSpec((2, BR, L), lambda i: (0, i, 0))],
        out_specs=[
            pl.BlockSpec((BR, 3), lambda i: (i, 0)),
            pl.BlockSpec((BR, 3), lambda i: (i, 0)),
        ],
        out_shape=[
            jax.ShapeDtypeStruct((Vp, 3), jnp.float32),
            jax.ShapeDtypeStruct((Vp, 3), jnp.float32),
        ],
    )(parts)


def kernel(base_bxyz, bcenter, e_base, e_voxel):
    f32 = jnp.float32
    N = base_bxyz.shape[0]
    V = bcenter.shape[0]
    E = e_base.shape[0]
    Vp = ((V + 1 + 127) // 128) * 128
    RS = Vp // NS
    NITER = E // NW // CH

    base_pad = jnp.concatenate(
        [base_bxyz.astype(f32),
         jnp.ones((N, 1), f32),
         jnp.zeros((N, L - 5), f32)], axis=1)
    eb4 = e_base.reshape(E // CH, KSUB, SUB)
    ev4 = e_voxel.reshape(E // CH, KSUB, SUB)
    zeros = jnp.zeros((RS, L), f32)
    ctab = jnp.array(_CTAB, jnp.int32)

    starts = _tc_starts(e_voxel.reshape(NW, E // NW))
    st = jnp.concatenate([starts, jnp.full((1,), E, jnp.int32)])
    starts_full = jnp.concatenate(
        [st[:NW, None], st[1:NW + 1, None],
         jnp.zeros((NW, L - 2), jnp.int32)], axis=1)

    parts1 = _make_sc_moments(E, V, NITER, Vp, RS)(
        base_pad, eb4, ev4, ctab, zeros)
    vol, bxyz, cov6 = _tc_stats(parts1, bcenter)
    mask = vol > 0.5
    c = cov6
    cov33 = jnp.stack(
        [c[:, 0], c[:, 1], c[:, 2],
         c[:, 1], c[:, 3], c[:, 4],
         c[:, 2], c[:, 4], c[:, 5]], axis=1).reshape(V, 3, 3)
    eigvals, eigvecs = jnp.linalg.eigh(cov33)
    voxpack = jnp.concatenate(
        [eigvecs.reshape(V, 9), bxyz[:, 1:4], jnp.zeros((V, 4), f32)],
        axis=1)
    parts3 = _make_sc_proj(E, V, Vp, RS)(
        base_pad, voxpack, eb4, ev4, starts_full, ctab, zeros)
    pmaxp, pminp = _tc_combine(parts3)
    return (bxyz, vol, mask, eigvals, eigvecs,
            pmaxp[:V], pminp[:V])


# speedup vs baseline: 95.7879x; 52.8800x over previous
import functools

import jax
import jax.numpy as jnp
from jax import lax
from jax.experimental import pallas as pl
from jax.experimental.pallas import tpu as pltpu
from jax.experimental.pallas import tpu_sc as plsc

L = 16
NC = 2
NS = 16
NW = NC * NS
SUB = 80
KSUB = 5
CH = SUB * KSUB

_CTAB = [
    [4, 0, 1, 2, 3, 1, 1, 1, 2, 2, 3, 5, 5, 5, 5, 5],
    [4, 4, 4, 4, 4, 1, 2, 3, 2, 3, 3, 5, 5, 5, 5, 5],
    [1, 1, 1, 2, 2, 2, 3, 3, 3, 5, 5, 5, 5, 5, 5, 5],
    [9, 9, 9, 10, 10, 10, 11, 11, 11, 12, 12, 12, 12, 12, 12, 12],
    [3, 4, 5, 6, 7, 8, 9, 10, 11, 12, 12, 12, 12, 12, 12, 12],
    [6, 7, 8, 9, 10, 11, 12, 12, 12, 12, 12, 12, 12, 12, 12, 12],
    [0, 1, 2, 0, 1, 2, 12, 12, 12, 12, 12, 12, 12, 12, 12, 12],
    [1, 1, 1, -1, -1, -1, 0, 0, 0, 0, 0, 0, 0, 0, 0, 0],
    [0, 0, 1, 2, 3, 4, 5, 6, 7, 8, 9, 10, 11, 12, 13, 14],
    [15] * 16,
    [1] + [0] * 15,
    [1, 2, 3, 4, 5, 6, 7, 8, 9, 10, 11, 12, 13, 14, 15, 15],
    [0] * 15 + [1],
]


def _lane():
    return lax.iota(jnp.int32, L)


def _make_sc_moments(E, V, NITER, Vp, RS):
    mesh = plsc.VectorSubcoreMesh(core_axis_name="c", subcore_axis_name="s")

    @functools.partial(
        pl.kernel,
        out_type=jax.ShapeDtypeStruct((NC, Vp, L), jnp.float32),
        mesh=mesh,
        compiler_params=pltpu.CompilerParams(use_tc_tiling_on_sc=False),
        scratch_types=[
            pltpu.VMEM_SHARED((Vp, L), jnp.float32),
            pltpu.VMEM((len(_CTAB), L), jnp.int32),
            pltpu.VMEM((KSUB, SUB), jnp.int32),
            pltpu.VMEM((KSUB, SUB), jnp.int32),
            pltpu.VMEM((CH, L), jnp.float32),
            pltpu.VMEM((CH, L), jnp.float32),
            pltpu.SemaphoreType.DMA,
        ],
    )
    def sc1(base_ref, eb_ref, ev_ref, ctab_ref, zeros_ref, out_ref,
            acc, ctab, idxb, idxv, rows, pay, sem):
        cid = lax.axis_index("c")
        sid = lax.axis_index("s")
        gw = cid * NS + sid
        r0 = sid * RS
        pltpu.sync_copy(ctab_ref, ctab)
        pltpu.sync_copy(zeros_ref, acc.at[pl.ds(r0, RS)])
        idxa_v = ctab[0]
        idxb_v = ctab[1]
        plsc.subcore_barrier()

        def it(i, carry):
            t = gw * NITER + i
            pltpu.sync_copy(eb_ref.at[t], idxb)
            pltpu.sync_copy(ev_ref.at[t], idxv)
            cps = [
                pltpu.async_copy(base_ref.at[idxb.at[k]],
                                 rows.at[pl.ds(k * SUB, SUB)], sem)
                for k in range(KSUB)
            ]
            for cp in cps:
                cp.wait()

            def grp(g, c):
                for u in range(L):
                    r = g * L + u
                    prow = rows[r]
                    a = jnp.take_along_axis(prow, idxa_v, axis=0)
                    b = jnp.take_along_axis(prow, idxb_v, axis=0)
                    pay[r] = a * b
                return c

            lax.fori_loop(0, CH // L, grp, 0)
            for k in range(KSUB):
                pltpu.sync_copy(pay.at[pl.ds(k * SUB, SUB)],
                                acc.at[idxv.at[k]], add=True)
            return carry

        lax.fori_loop(0, NITER, it, 0)
        plsc.subcore_barrier()
        pltpu.sync_copy(acc.at[pl.ds(r0, RS)], out_ref.at[cid, pl.ds(r0, RS)])

    return sc1


def _make_sc_proj(E, V, Vp, RS):
    mesh = plsc.VectorSubcoreMesh(core_axis_name="c", subcore_axis_name="s")
    TCH = E // CH
    DUMMY = V

    @functools.partial(
        pl.kernel,
        out_type=jax.ShapeDtypeStruct((NC, Vp, L), jnp.float32),
        mesh=mesh,
        compiler_params=pltpu.CompilerParams(use_tc_tiling_on_sc=False),
        scratch_types=[
            pltpu.VMEM_SHARED((Vp, L), jnp.float32),
            pltpu.VMEM((len(_CTAB), L), jnp.int32),
            pltpu.VMEM((L,), jnp.int32),
            pltpu.VMEM((KSUB, SUB), jnp.int32),
            pltpu.VMEM((KSUB, SUB), jnp.int32),
            pltpu.VMEM((KSUB, SUB), jnp.int32),
            pltpu.VMEM((CH, L), jnp.float32),
            pltpu.VMEM((CH, L), jnp.float32),
            pltpu.VMEM((CH, L), jnp.float32),
            pltpu.VMEM((L,), jnp.int32),
            pltpu.VMEM((L,), jnp.float32),
            pltpu.SemaphoreType.DMA,
        ],
    )
    def sc3(base_ref, vox_ref, eb_ref, ev_ref, starts_ref, ctab_ref,
            zeros_ref, out_ref,
            acc, ctab, starts, idxb, idxv, sidx, rows, vrows, pay,
            prevbuf, rbuf, sem):
        cid = lax.axis_index("c")
        sid = lax.axis_index("s")
        gw = cid * NS + sid
        lane = _lane()
        r0 = sid * RS
        pltpu.sync_copy(ctab_ref, ctab)
        pltpu.sync_copy(starts_ref.at[gw], starts)
        pltpu.sync_copy(zeros_ref, acc.at[pl.ds(r0, RS)])
        idxp_v = ctab[2]
        idxm_v = ctab[3]
        sh3_v = ctab[4]
        sh6_v = ctab[5]
        packf_v = ctab[6]
        signf = ctab[7].astype(jnp.float32)
        shd1_v = ctab[8]
        spl15_v = ctab[9]
        ind0_v = ctab[10]
        shu1_v = ctab[11]
        ind15_f = ctab[12].astype(jnp.float32)
        zsplat = jnp.zeros((L,), jnp.int32)
        sev = starts[...]
        sv = sev[0]
        es = sev[1]
        t0 = sv // CH
        prevbuf[...] = jnp.full((L,), -1, jnp.int32)
        rbuf[...] = jnp.zeros((L,), jnp.float32)
        plsc.subcore_barrier()

        def chunk(i, carry):
            t = t0 + i

            @pl.when(t * CH < es)
            def _():
                pltpu.sync_copy(eb_ref.at[t], idxb)
                pltpu.sync_copy(ev_ref.at[t], idxv)
                cps = []
                for k in range(KSUB):
                    cps.append(pltpu.async_copy(
                        base_ref.at[idxb.at[k]],
                        rows.at[pl.ds(k * SUB, SUB)], sem))
                    cps.append(pltpu.async_copy(
                        vox_ref.at[idxv.at[k]],
                        vrows.at[pl.ds(k * SUB, SUB)], sem))
                for cp in cps:
                    cp.wait()

                def fin(gp, vvp, okp, samep, nextsame, mgf):
                    shifted = jnp.take_along_axis(samep, shu1_v, axis=0)
                    nxt = shifted * (1.0 - ind15_f) + nextsame * ind15_f
                    ki = ((1.0 - nxt) * okp.astype(jnp.float32) * mgf
                          ).astype(jnp.int32)
                    kkp = gp // KSUB
                    jjp = (gp % KSUB) * L
                    sidx[kkp, pl.ds(jjp, L)] = (vvp * ki
                                                + DUMMY * (1 - ki))

                def grp(g, st):
                    prevv, R, samep, vvp, okp = st
                    kk = g // KSUB
                    jj = (g % KSUB) * L
                    vv = idxv[kk, pl.ds(jj, L)]
                    sh = jnp.take_along_axis(vv, shd1_v, axis=0)
                    sh0 = prevv * ind0_v + sh * (1 - ind0_v)
                    same_f = (1 - jnp.minimum(jnp.abs(vv - sh0), 1)
                              ).astype(jnp.float32)
                    gi = jnp.full((L,), t * CH + g * L, jnp.int32) + lane
                    okv = (jnp.minimum(jnp.maximum(gi - sv + 1, 0), 1)
                           * jnp.minimum(jnp.maximum(es - gi, 0), 1))
                    firstg = jnp.take_along_axis(same_f, zsplat, axis=0)
                    mgf = jnp.minimum(g, 1).astype(jnp.float32)
                    fin(jnp.maximum(g - 1, 0), vvp, okp, samep, firstg, mgf)
                    for u in range(L):
                        r = g * L + u
                        prow = rows[r]
                        vrow = vrows[r]
                        dv = (jnp.take_along_axis(prow, idxp_v, axis=0)
                              - jnp.take_along_axis(vrow, idxm_v, axis=0))
                        q = dv * vrow
                        ss = (q + jnp.take_along_axis(q, sh3_v, axis=0)
                              + jnp.take_along_axis(q, sh6_v, axis=0))
                        f = jnp.take_along_axis(ss, packf_v, axis=0) * signf
                        su = jnp.take_along_axis(
                            same_f, jnp.full((L,), u, jnp.int32), axis=0)
                        R = su * jnp.maximum(R, f) + (1.0 - su) * f
                        pay[r] = R * signf
                    prev2 = jnp.take_along_axis(vv, spl15_v, axis=0)
                    return (prev2, R, same_f, vv, okv)

                st = lax.fori_loop(
                    0, CH // L, grp,
                    (prevbuf[...], rbuf[...], jnp.zeros((L,), jnp.float32),
                     jnp.full((L,), -1, jnp.int32), jnp.zeros((L,), jnp.int32)))
                prevbuf[...] = st[0]
                rbuf[...] = st[1]
                fin(CH // L - 1, st[3], st[4], st[2],
                    jnp.zeros((L,), jnp.float32), jnp.float32(1.0))
                for k in range(KSUB):
                    pltpu.sync_copy(pay.at[pl.ds(k * SUB, SUB)],
                                    acc.at[sidx.at[k]])

            return carry

        lax.fori_loop(0, TCH, chunk, 0)
        plsc.subcore_barrier()
        pltpu.sync_copy(acc.at[pl.ds(r0, RS)], out_ref.at[cid, pl.ds(r0, RS)])

    return sc3


def _tc_starts(ev2):
    Ew = ev2.shape[1]
    E = NW * Ew

    def body(ev_ref, out_ref):
        v2 = ev_ref[...]
        head = jnp.concatenate([v2[:1, :1] - 1, v2[:-1, -1:]], axis=0)
        prev2 = jnp.concatenate([head, v2[:, :-1]], axis=1)
        bound = v2 != prev2
        gidx = (lax.broadcasted_iota(jnp.int32, (NW, Ew), 0) * Ew
                + lax.broadcasted_iota(jnp.int32, (NW, Ew), 1))
        idxs = jnp.where(bound, gidx, E)
        bmin = jnp.min(idxs, axis=1)
        i32 = lax.iota(jnp.int32, NW)
        suf = jnp.min(
            jnp.where(i32[None, :] >= i32[:, None], bmin[None, :], E),
            axis=1)
        out_ref[...] = suf

    return pl.pallas_call(
        body,
        in_specs=[pl.BlockSpec(memory_space=pltpu.VMEM)],
        out_specs=pl.BlockSpec(memory_space=pltpu.VMEM),
        out_shape=jax.ShapeDtypeStruct((NW,), jnp.int32),
    )(ev2)


def _tc_stats(parts, bcenter):
    V = bcenter.shape[0]
    BR = 512
    grid = (pl.cdiv(V, BR),)
    SWEEPS = 8
    ORDER = ((0, 2), (1, 2), (0, 1))

    def body(p_ref, c_ref, vol_ref, bxyz_ref, w_ref, v_ref, pack_ref):
        p = p_ref[0] + p_ref[1]
        n = p[:, 0]
        safe = jnp.maximum(n, 1.0)
        mean4 = p[:, 1:5] / safe[:, None]
        mask = n > 0.5
        bxyz = jnp.where(mask[:, None], mean4, c_ref[...])
        bxyz_ref[...] = bxyz
        vol_ref[...] = n
        mx, my, mz = mean4[:, 1], mean4[:, 2], mean4[:, 3]
        s2 = p[:, 5:11] / safe[:, None]

        a = {(0, 0): s2[:, 0] - mx * mx, (0, 1): s2[:, 1] - mx * my,
             (0, 2): s2[:, 2] - mx * mz, (1, 1): s2[:, 3] - my * my,
             (1, 2): s2[:, 4] - my * mz, (2, 2): s2[:, 5] - mz * mz}
        one = jnp.ones_like(a[(0, 0)])
        zero = jnp.zeros_like(one)
        vcols = [[one, zero, zero], [zero, one, zero], [zero, zero, one]]

        def ga(i, j):
            return a[(i, j)] if i <= j else a[(j, i)]

        def sa(i, j, val):
            a[(i, j) if i <= j else (j, i)] = val

        for _ in range(SWEEPS):
            for (pp, qq) in ORDER:
                rr = 3 - pp - qq
                apq = ga(pp, qq)
                app = ga(pp, pp)
                aqq = ga(qq, qq)
                tau = (aqq - app) / (2.0 * apq)
                t = jnp.sign(tau) / (jnp.abs(tau) + jnp.sqrt(1.0 + tau * tau))
                t = jnp.where(apq == 0.0, 0.0, t)
                c = 1.0 / jnp.sqrt(1.0 + t * t)
                s = t * c
                apr = ga(pp, rr)
                aqr = ga(qq, rr)
                sa(pp, pp, c * (c * app - s * apq) - s * (c * apq - s * aqq))
                sa(qq, qq, s * (s * app + c * apq) + c * (s * apq + c * aqq))
                sa(pp, qq, c * (s * app + c * apq) - s * (s * apq + c * aqq))
                sa(pp, rr, c * apr - s * aqr)
                sa(qq, rr, s * apr + c * aqr)
                for row in range(3):
                    vp = vcols[row][pp]
                    vq = vcols[row][qq]
                    vcols[row][pp] = c * vp - s * vq
                    vcols[row][qq] = s * vp + c * vq

        w = [ga(0, 0), ga(1, 1), ga(2, 2)]
        r0 = ((w[1] < w[0]).astype(jnp.int32)
              + (w[2] < w[0]).astype(jnp.int32))
        r1 = ((w[0] <= w[1]).astype(jnp.int32)
              + (w[2] < w[1]).astype(jnp.int32))
        r2 = ((w[0] <= w[2]).astype(jnp.int32)
              + (w[1] <= w[2]).astype(jnp.int32))
        ranks = [r0, r1, r2]

        def pick(vals, k):
            out = jnp.zeros_like(vals[0])
            for j in range(3):
                out = jnp.where(ranks[j] == k, vals[j], out)
            return out

        ws = [pick(w, k) for k in range(3)]
        w_ref[...] = jnp.stack(ws, axis=1)
        vs = [[pick(vcols[row], k) for k in range(3)] for row in range(3)]
        flat = [vs[row][k] for row in range(3) for k in range(3)]
        v_ref[...] = jnp.stack(flat, axis=1)
        pack_ref[...] = jnp.stack(
            flat + [bxyz[:, 1], bxyz[:, 2], bxyz[:, 3],
                    zero, zero, zero, zero], axis=1)

    return pl.pallas_call(
        body,
        grid=grid,
        in_specs=[
            pl.BlockSpec((2, BR, L), lambda i: (0, i, 0)),
            pl.BlockSpec((BR, 4), lambda i: (i, 0)),
        ],
        out_specs=[
            pl.BlockSpec((BR,), lambda i: (i,)),
            pl.BlockSpec((BR, 4), lambda i: (i, 0)),
            pl.BlockSpec((BR, 3), lambda i: (i, 0)),
            pl.BlockSpec((BR, 9), lambda i: (i, 0)),
            pl.BlockSpec((BR, L), lambda i: (i, 0)),
        ],
        out_shape=[
            jax.ShapeDtypeStruct((V,), jnp.float32),
            jax.ShapeDtypeStruct((V, 4), jnp.float32),
            jax.ShapeDtypeStruct((V, 3), jnp.float32),
            jax.ShapeDtypeStruct((V, 9), jnp.float32),
            jax.ShapeDtypeStruct((V, L), jnp.float32),
        ],
    )(parts, bcenter)


def _tc_combine(parts):
    Vp = parts.shape[1]
    BR = 2048
    grid = (pl.cdiv(Vp, BR),)

    def body(p_ref, mx_ref, mn_ref):
        p = p_ref[0] + p_ref[1]
        mx_ref[...] = p[:, 0:3]
        mn_ref[...] = p[:, 3:6]

    return pl.pallas_call(
        body,
        grid=grid,
        in_specs=[pl.BlockSpec((2, BR, L), lambda i: (0, i, 0))],
        out_specs=[
            pl.BlockSpec((BR, 3), lambda i: (i, 0)),
            pl.BlockSpec((BR, 3), lambda i: (i, 0)),
        ],
        out_shape=[
            jax.ShapeDtypeStruct((Vp, 3), jnp.float32),
            jax.ShapeDtypeStruct((Vp, 3), jnp.float32),
        ],
    )(parts)


def kernel(base_bxyz, bcenter, e_base, e_voxel):
    f32 = jnp.float32
    N = base_bxyz.shape[0]
    V = bcenter.shape[0]
    E = e_base.shape[0]
    Vp = ((V + 1 + 127) // 128) * 128
    RS = Vp // NS
    NITER = E // NW // CH

    base_pad = jnp.concatenate(
        [base_bxyz.astype(f32),
         jnp.ones((N, 1), f32),
         jnp.zeros((N, L - 5), f32)], axis=1)
    eb4 = e_base.reshape(E // CH, KSUB, SUB)
    ev4 = e_voxel.reshape(E // CH, KSUB, SUB)
    zeros = jnp.zeros((RS, L), f32)
    ctab = jnp.array(_CTAB, jnp.int32)

    starts = _tc_starts(e_voxel.reshape(NW, E // NW))
    st = jnp.concatenate([starts, jnp.full((1,), E, jnp.int32)])
    starts_full = jnp.concatenate(
        [st[:NW, None], st[1:NW + 1, None],
         jnp.zeros((NW, L - 2), jnp.int32)], axis=1)

    parts1 = _make_sc_moments(E, V, NITER, Vp, RS)(
        base_pad, eb4, ev4, ctab, zeros)
    vol, bxyz, eigvals, eigv9, voxpack = _tc_stats(parts1, bcenter)
    mask = vol > 0.5
    eigvecs = eigv9.reshape(V, 3, 3)
    parts3 = _make_sc_proj(E, V, Vp, RS)(
        base_pad, voxpack, eb4, ev4, starts_full, ctab, zeros)
    pmaxp, pminp = _tc_combine(parts3)
    return (bxyz, vol, mask, eigvals, eigvecs,
            pmaxp[:V], pminp[:V])


# speedup vs baseline: 139.7791x; 1.4593x over previous
import functools

import jax
import jax.numpy as jnp
from jax import lax
from jax.experimental import pallas as pl
from jax.experimental.pallas import tpu as pltpu
from jax.experimental.pallas import tpu_sc as plsc

L = 16
NC = 2
NS = 16
NW = NC * NS
SUB = 80
KSUB = 5
CH = SUB * KSUB

_CTAB = [
    [4, 0, 1, 2, 3, 1, 1, 1, 2, 2, 3, 5, 5, 5, 5, 5],
    [4, 4, 4, 4, 4, 1, 2, 3, 2, 3, 3, 5, 5, 5, 5, 5],
    [1, 1, 1, 2, 2, 2, 3, 3, 3, 5, 5, 5, 5, 5, 5, 5],
    [9, 9, 9, 10, 10, 10, 11, 11, 11, 12, 12, 12, 12, 12, 12, 12],
    [3, 4, 5, 6, 7, 8, 9, 10, 11, 12, 12, 12, 12, 12, 12, 12],
    [6, 7, 8, 9, 10, 11, 12, 12, 12, 12, 12, 12, 12, 12, 12, 12],
    [0, 1, 2, 0, 1, 2, 12, 12, 12, 12, 12, 12, 12, 12, 12, 12],
    [1, 1, 1, -1, -1, -1, 0, 0, 0, 0, 0, 0, 0, 0, 0, 0],
    [0, 0, 1, 2, 3, 4, 5, 6, 7, 8, 9, 10, 11, 12, 13, 14],
    [15] * 16,
    [1] + [0] * 15,
    [1, 2, 3, 4, 5, 6, 7, 8, 9, 10, 11, 12, 13, 14, 15, 15],
    [0] * 15 + [1],
]


def _lane():
    return lax.iota(jnp.int32, L)


def _make_sc_moments(E, V, NITER, Vp, RS):
    mesh = plsc.VectorSubcoreMesh(core_axis_name="c", subcore_axis_name="s")

    @functools.partial(
        pl.kernel,
        out_type=jax.ShapeDtypeStruct((NC, Vp, L), jnp.float32),
        mesh=mesh,
        compiler_params=pltpu.CompilerParams(use_tc_tiling_on_sc=False),
        scratch_types=[
            pltpu.VMEM_SHARED((Vp, L), jnp.float32),
            pltpu.VMEM((len(_CTAB), L), jnp.int32),
            pltpu.VMEM((KSUB, SUB), jnp.int32),
            pltpu.VMEM((KSUB, SUB), jnp.int32),
            pltpu.VMEM((CH, L), jnp.float32),
            pltpu.VMEM((CH, L), jnp.float32),
            pltpu.SemaphoreType.DMA,
        ],
    )
    def sc1(base_ref, eb_ref, ev_ref, ctab_ref, zeros_ref, out_ref,
            acc, ctab, idxb, idxv, rows, pay, sem):
        cid = lax.axis_index("c")
        sid = lax.axis_index("s")
        gw = cid * NS + sid
        r0 = sid * RS
        pltpu.sync_copy(ctab_ref, ctab)
        pltpu.sync_copy(zeros_ref, acc.at[pl.ds(r0, RS)])
        idxa_v = ctab[0]
        idxb_v = ctab[1]
        plsc.subcore_barrier()

        def it(i, carry):
            t = gw * NITER + i
            pltpu.sync_copy(eb_ref.at[t], idxb)
            pltpu.sync_copy(ev_ref.at[t], idxv)
            cps = [
                pltpu.async_copy(base_ref.at[idxb.at[k]],
                                 rows.at[pl.ds(k * SUB, SUB)], sem)
                for k in range(KSUB)
            ]
            for cp in cps:
                cp.wait()

            def grp(g, c):
                for u in range(L):
                    r = g * L + u
                    prow = rows[r]
                    a = jnp.take_along_axis(prow, idxa_v, axis=0)
                    b = jnp.take_along_axis(prow, idxb_v, axis=0)
                    pay[r] = a * b
                return c

            lax.fori_loop(0, CH // L, grp, 0)
            for k in range(KSUB):
                pltpu.sync_copy(pay.at[pl.ds(k * SUB, SUB)],
                                acc.at[idxv.at[k]], add=True)
            return carry

        lax.fori_loop(0, NITER, it, 0)
        plsc.subcore_barrier()
        pltpu.sync_copy(acc.at[pl.ds(r0, RS)], out_ref.at[cid, pl.ds(r0, RS)])

    return sc1


def _make_sc_proj(E, V, Vp, RS):
    mesh = plsc.VectorSubcoreMesh(core_axis_name="c", subcore_axis_name="s")
    TCH = E // CH
    DUMMY = V

    @functools.partial(
        pl.kernel,
        out_type=jax.ShapeDtypeStruct((NC, Vp, L), jnp.float32),
        mesh=mesh,
        compiler_params=pltpu.CompilerParams(use_tc_tiling_on_sc=False),
        scratch_types=[
            pltpu.VMEM_SHARED((Vp, L), jnp.float32),
            pltpu.VMEM((len(_CTAB), L), jnp.int32),
            pltpu.VMEM((L,), jnp.int32),
            pltpu.VMEM((KSUB, SUB), jnp.int32),
            pltpu.VMEM((KSUB, SUB), jnp.int32),
            pltpu.VMEM((KSUB, SUB), jnp.int32),
            pltpu.VMEM((CH, L), jnp.float32),
            pltpu.VMEM((CH, L), jnp.float32),
            pltpu.VMEM((CH, L), jnp.float32),
            pltpu.VMEM((L,), jnp.int32),
            pltpu.VMEM((L,), jnp.float32),
            pltpu.SemaphoreType.DMA,
        ],
    )
    def sc3(base_ref, vox_ref, eb_ref, ev_ref, starts_ref, ctab_ref,
            zeros_ref, out_ref,
            acc, ctab, starts, idxb, idxv, sidx, rows, vrows, pay,
            prevbuf, rbuf, sem):
        cid = lax.axis_index("c")
        sid = lax.axis_index("s")
        gw = cid * NS + sid
        lane = _lane()
        r0 = sid * RS
        pltpu.sync_copy(ctab_ref, ctab)
        pltpu.sync_copy(starts_ref.at[gw], starts)
        pltpu.sync_copy(zeros_ref, acc.at[pl.ds(r0, RS)])
        idxp_v = ctab[2]
        idxm_v = ctab[3]
        sh3_v = ctab[4]
        sh6_v = ctab[5]
        packf_v = ctab[6]
        signf = ctab[7].astype(jnp.float32)
        shd1_v = ctab[8]
        spl15_v = ctab[9]
        ind0_v = ctab[10]
        shu1_v = ctab[11]
        ind15_f = ctab[12].astype(jnp.float32)
        zsplat = jnp.zeros((L,), jnp.int32)
        sev = starts[...]
        sv = sev[0]
        es = sev[1]
        t0 = sv // CH
        prevbuf[...] = jnp.full((L,), -1, jnp.int32)
        rbuf[...] = jnp.zeros((L,), jnp.float32)
        plsc.subcore_barrier()

        def chunk(i, carry):
            t = t0 + i

            @pl.when(t * CH < es)
            def _():
                pltpu.sync_copy(eb_ref.at[t], idxb)
                pltpu.sync_copy(ev_ref.at[t], idxv)
                cps = []
                for k in range(KSUB):
                    cps.append(pltpu.async_copy(
                        base_ref.at[idxb.at[k]],
                        rows.at[pl.ds(k * SUB, SUB)], sem))
                    cps.append(pltpu.async_copy(
                        vox_ref.at[idxv.at[k]],
                        vrows.at[pl.ds(k * SUB, SUB)], sem))
                for cp in cps:
                    cp.wait()

                def fin(gp, vvp, okp, samep, nextsame, mgf):
                    shifted = jnp.take_along_axis(samep, shu1_v, axis=0)
                    nxt = shifted * (1.0 - ind15_f) + nextsame * ind15_f
                    ki = ((1.0 - nxt) * okp.astype(jnp.float32) * mgf
                          ).astype(jnp.int32)
                    kkp = gp // KSUB
                    jjp = (gp % KSUB) * L
                    sidx[kkp, pl.ds(jjp, L)] = (vvp * ki
                                                + DUMMY * (1 - ki))

                def grp(g, st):
                    prevv, R, samep, vvp, okp = st
                    kk = g // KSUB
                    jj = (g % KSUB) * L
                    vv = idxv[kk, pl.ds(jj, L)]
                    sh = jnp.take_along_axis(vv, shd1_v, axis=0)
                    sh0 = prevv * ind0_v + sh * (1 - ind0_v)
                    same_f = (1 - jnp.minimum(jnp.abs(vv - sh0), 1)
                              ).astype(jnp.float32)
                    gi = jnp.full((L,), t * CH + g * L, jnp.int32) + lane
                    okv = (jnp.minimum(jnp.maximum(gi - sv + 1, 0), 1)
                           * jnp.minimum(jnp.maximum(es - gi, 0), 1))
                    firstg = jnp.take_along_axis(same_f, zsplat, axis=0)
                    mgf = jnp.minimum(g, 1).astype(jnp.float32)
                    fin(jnp.maximum(g - 1, 0), vvp, okp, samep, firstg, mgf)
                    for u in range(L):
                        r = g * L + u
                        prow = rows[r]
                        vrow = vrows[r]
                        dv = (jnp.take_along_axis(prow, idxp_v, axis=0)
                              - jnp.take_along_axis(vrow, idxm_v, axis=0))
                        q = dv * vrow
                        ss = (q + jnp.take_along_axis(q, sh3_v, axis=0)
                              + jnp.take_along_axis(q, sh6_v, axis=0))
                        f = jnp.take_along_axis(ss, packf_v, axis=0) * signf
                        su = jnp.take_along_axis(
                            same_f, jnp.full((L,), u, jnp.int32), axis=0)
                        R = su * jnp.maximum(R, f) + (1.0 - su) * f
                        pay[r] = R * signf
                    prev2 = jnp.take_along_axis(vv, spl15_v, axis=0)
                    return (prev2, R, same_f, vv, okv)

                st = lax.fori_loop(
                    0, CH // L, grp,
                    (prevbuf[...], rbuf[...], jnp.zeros((L,), jnp.float32),
                     jnp.full((L,), -1, jnp.int32), jnp.zeros((L,), jnp.int32)))
                prevbuf[...] = st[0]
                rbuf[...] = st[1]
                fin(CH // L - 1, st[3], st[4], st[2],
                    jnp.zeros((L,), jnp.float32), jnp.float32(1.0))
                for k in range(KSUB):
                    pltpu.sync_copy(pay.at[pl.ds(k * SUB, SUB)],
                                    acc.at[sidx.at[k]])

            return carry

        lax.fori_loop(0, TCH, chunk, 0)
        plsc.subcore_barrier()
        pltpu.sync_copy(acc.at[pl.ds(r0, RS)], out_ref.at[cid, pl.ds(r0, RS)])

    return sc3


def _tc_starts(ev2):
    Ew = ev2.shape[1]
    E = NW * Ew

    def body(ev_ref, out_ref):
        v2 = ev_ref[...]
        head = jnp.concatenate([v2[:1, :1] - 1, v2[:-1, -1:]], axis=0)
        prev2 = jnp.concatenate([head, v2[:, :-1]], axis=1)
        bound = v2 != prev2
        gidx = (lax.broadcasted_iota(jnp.int32, (NW, Ew), 0) * Ew
                + lax.broadcasted_iota(jnp.int32, (NW, Ew), 1))
        idxs = jnp.where(bound, gidx, E)
        bmin = jnp.min(idxs, axis=1)
        i32 = lax.iota(jnp.int32, NW)
        suf = jnp.min(
            jnp.where(i32[None, :] >= i32[:, None], bmin[None, :], E),
            axis=1)
        out_ref[...] = suf

    return pl.pallas_call(
        body,
        in_specs=[pl.BlockSpec(memory_space=pltpu.VMEM)],
        out_specs=pl.BlockSpec(memory_space=pltpu.VMEM),
        out_shape=jax.ShapeDtypeStruct((NW,), jnp.int32),
    )(ev2)


def _tc_stats(parts, bcenter):
    V = bcenter.shape[0]
    BR = 1024
    grid = (pl.cdiv(V, BR),)
    SWEEPS = 4
    ORDER = ((0, 2), (1, 2), (0, 1))

    def body(p_ref, c_ref, vol_ref, bxyz_ref, w_ref, v_ref, pack_ref):
        p = p_ref[0] + p_ref[1]
        n = p[:, 0]
        safe = jnp.maximum(n, 1.0)
        mean4 = p[:, 1:5] / safe[:, None]
        mask = n > 0.5
        bxyz = jnp.where(mask[:, None], mean4, c_ref[...])
        bxyz_ref[...] = bxyz
        vol_ref[...] = n
        mx, my, mz = mean4[:, 1], mean4[:, 2], mean4[:, 3]
        s2 = p[:, 5:11] / safe[:, None]

        a = {(0, 0): s2[:, 0] - mx * mx, (0, 1): s2[:, 1] - mx * my,
             (0, 2): s2[:, 2] - mx * mz, (1, 1): s2[:, 3] - my * my,
             (1, 2): s2[:, 4] - my * mz, (2, 2): s2[:, 5] - mz * mz}
        one = jnp.ones_like(a[(0, 0)])
        zero = jnp.zeros_like(one)
        vcols = [[one, zero, zero], [zero, one, zero], [zero, zero, one]]

        def ga(i, j):
            return a[(i, j)] if i <= j else a[(j, i)]

        def sa(i, j, val):
            a[(i, j) if i <= j else (j, i)] = val

        for _ in range(SWEEPS):
            for (pp, qq) in ORDER:
                rr = 3 - pp - qq
                apq = ga(pp, qq)
                app = ga(pp, pp)
                aqq = ga(qq, qq)
                tau = (aqq - app) / (2.0 * apq)
                t = jnp.sign(tau) / (jnp.abs(tau) + jnp.sqrt(1.0 + tau * tau))
                t = jnp.where(apq == 0.0, 0.0, t)
                c = 1.0 / jnp.sqrt(1.0 + t * t)
                s = t * c
                apr = ga(pp, rr)
                aqr = ga(qq, rr)
                sa(pp, pp, c * (c * app - s * apq) - s * (c * apq - s * aqq))
                sa(qq, qq, s * (s * app + c * apq) + c * (s * apq + c * aqq))
                sa(pp, qq, c * (s * app + c * apq) - s * (s * apq + c * aqq))
                sa(pp, rr, c * apr - s * aqr)
                sa(qq, rr, s * apr + c * aqr)
                for row in range(3):
                    vp = vcols[row][pp]
                    vq = vcols[row][qq]
                    vcols[row][pp] = c * vp - s * vq
                    vcols[row][qq] = s * vp + c * vq

        w = [ga(0, 0), ga(1, 1), ga(2, 2)]
        r0 = ((w[1] < w[0]).astype(jnp.int32)
              + (w[2] < w[0]).astype(jnp.int32))
        r1 = ((w[0] <= w[1]).astype(jnp.int32)
              + (w[2] < w[1]).astype(jnp.int32))
        r2 = ((w[0] <= w[2]).astype(jnp.int32)
              + (w[1] <= w[2]).astype(jnp.int32))
        ranks = [r0, r1, r2]

        def pick(vals, k):
            out = jnp.zeros_like(vals[0])
            for j in range(3):
                out = jnp.where(ranks[j] == k, vals[j], out)
            return out

        ws = [pick(w, k) for k in range(3)]
        w_ref[...] = jnp.stack(ws, axis=1)
        vs = [[pick(vcols[row], k) for k in range(3)] for row in range(3)]
        flat = [vs[row][k] for row in range(3) for k in range(3)]
        v_ref[...] = jnp.stack(flat, axis=1)
        pack_ref[...] = jnp.stack(
            flat + [bxyz[:, 1], bxyz[:, 2], bxyz[:, 3],
                    zero, zero, zero, zero], axis=1)

    return pl.pallas_call(
        body,
        grid=grid,
        in_specs=[
            pl.BlockSpec((2, BR, L), lambda i: (0, i, 0)),
            pl.BlockSpec((BR, 4), lambda i: (i, 0)),
        ],
        out_specs=[
            pl.BlockSpec((BR,), lambda i: (i,)),
            pl.BlockSpec((BR, 4), lambda i: (i, 0)),
            pl.BlockSpec((BR, 3), lambda i: (i, 0)),
            pl.BlockSpec((BR, 9), lambda i: (i, 0)),
            pl.BlockSpec((BR, L), lambda i: (i, 0)),
        ],
        out_shape=[
            jax.ShapeDtypeStruct((V,), jnp.float32),
            jax.ShapeDtypeStruct((V, 4), jnp.float32),
            jax.ShapeDtypeStruct((V, 3), jnp.float32),
            jax.ShapeDtypeStruct((V, 9), jnp.float32),
            jax.ShapeDtypeStruct((V, L), jnp.float32),
        ],
    )(parts, bcenter)


def _tc_combine(parts):
    Vp = parts.shape[1]
    BR = 2048
    grid = (pl.cdiv(Vp, BR),)

    def body(p_ref, mx_ref, mn_ref):
        p = p_ref[0] + p_ref[1]
        mx_ref[...] = p[:, 0:3]
        mn_ref[...] = p[:, 3:6]

    return pl.pallas_call(
        body,
        grid=grid,
        in_specs=[pl.BlockSpec((2, BR, L), lambda i: (0, i, 0))],
        out_specs=[
            pl.BlockSpec((BR, 3), lambda i: (i, 0)),
            pl.BlockSpec((BR, 3), lambda i: (i, 0)),
        ],
        out_shape=[
            jax.ShapeDtypeStruct((Vp, 3), jnp.float32),
            jax.ShapeDtypeStruct((Vp, 3), jnp.float32),
        ],
    )(parts)


def kernel(base_bxyz, bcenter, e_base, e_voxel):
    f32 = jnp.float32
    N = base_bxyz.shape[0]
    V = bcenter.shape[0]
    E = e_base.shape[0]
    Vp = ((V + 1 + 127) // 128) * 128
    RS = Vp // NS
    NITER = E // NW // CH

    base_pad = jnp.concatenate(
        [base_bxyz.astype(f32),
         jnp.ones((N, 1), f32),
         jnp.zeros((N, L - 5), f32)], axis=1)
    eb4 = e_base.reshape(E // CH, KSUB, SUB)
    ev4 = e_voxel.reshape(E // CH, KSUB, SUB)
    zeros = jnp.zeros((RS, L), f32)
    ctab = jnp.array(_CTAB, jnp.int32)

    starts = _tc_starts(e_voxel.reshape(NW, E // NW))
    st = jnp.concatenate([starts, jnp.full((1,), E, jnp.int32)])
    starts_full = jnp.concatenate(
        [st[:NW, None], st[1:NW + 1, None],
         jnp.zeros((NW, L - 2), jnp.int32)], axis=1)

    parts1 = _make_sc_moments(E, V, NITER, Vp, RS)(
        base_pad, eb4, ev4, ctab, zeros)
    vol, bxyz, eigvals, eigv9, voxpack = _tc_stats(parts1, bcenter)
    mask = vol > 0.5
    eigvecs = eigv9.reshape(V, 3, 3)
    parts3 = _make_sc_proj(E, V, Vp, RS)(
        base_pad, voxpack, eb4, ev4, starts_full, ctab, zeros)
    pmaxp, pminp = _tc_combine(parts3)
    return (bxyz, vol, mask, eigvals, eigvecs,
            pmaxp[:V], pminp[:V])


# speedup vs baseline: 155.9185x; 1.1155x over previous
import functools

import jax
import jax.numpy as jnp
from jax import lax
from jax.experimental import pallas as pl
from jax.experimental.pallas import tpu as pltpu
from jax.experimental.pallas import tpu_sc as plsc

L = 16
NC = 2
NS = 16
NW = NC * NS
SUB = 80
KSUB = 5
CH = SUB * KSUB

_CTAB = [
    [4, 0, 1, 2, 3, 1, 1, 1, 2, 2, 3, 5, 5, 5, 5, 5],
    [4, 4, 4, 4, 4, 1, 2, 3, 2, 3, 3, 5, 5, 5, 5, 5],
    [1, 1, 1, 2, 2, 2, 3, 3, 3, 5, 5, 5, 5, 5, 5, 5],
    [9, 9, 9, 10, 10, 10, 11, 11, 11, 12, 12, 12, 12, 12, 12, 12],
    [3, 4, 5, 6, 7, 8, 9, 10, 11, 12, 12, 12, 12, 12, 12, 12],
    [6, 7, 8, 9, 10, 11, 12, 12, 12, 12, 12, 12, 12, 12, 12, 12],
    [0, 1, 2, 0, 1, 2, 12, 12, 12, 12, 12, 12, 12, 12, 12, 12],
    [1, 1, 1, -1, -1, -1, 0, 0, 0, 0, 0, 0, 0, 0, 0, 0],
    [0, 0, 1, 2, 3, 4, 5, 6, 7, 8, 9, 10, 11, 12, 13, 14],
    [15] * 16,
    [1] + [0] * 15,
    [1, 2, 3, 4, 5, 6, 7, 8, 9, 10, 11, 12, 13, 14, 15, 15],
    [0] * 15 + [1],
]


def _lane():
    return lax.iota(jnp.int32, L)


def _make_sc_moments(E, V, NITER, Vp, RS):
    mesh = plsc.VectorSubcoreMesh(core_axis_name="c", subcore_axis_name="s")

    @functools.partial(
        pl.kernel,
        out_type=jax.ShapeDtypeStruct((NC, Vp, L), jnp.float32),
        mesh=mesh,
        compiler_params=pltpu.CompilerParams(use_tc_tiling_on_sc=False),
        scratch_types=[
            pltpu.VMEM_SHARED((Vp, L), jnp.float32),
            pltpu.VMEM((len(_CTAB), L), jnp.int32),
            pltpu.VMEM((KSUB, SUB), jnp.int32),
            pltpu.VMEM((KSUB, SUB), jnp.int32),
            pltpu.VMEM((CH, L), jnp.float32),
            pltpu.VMEM((CH, L), jnp.float32),
            pltpu.SemaphoreType.DMA,
        ],
    )
    def sc1(base_ref, eb_ref, ev_ref, ctab_ref, zeros_ref, out_ref,
            acc, ctab, idxb, idxv, rows, pay, sem):
        cid = lax.axis_index("c")
        sid = lax.axis_index("s")
        gw = cid * NS + sid
        r0 = sid * RS
        pltpu.sync_copy(ctab_ref, ctab)
        pltpu.sync_copy(zeros_ref, acc.at[pl.ds(r0, RS)])
        idxa_v = ctab[0]
        idxb_v = ctab[1]
        plsc.subcore_barrier()

        def it(i, carry):
            t = gw * NITER + i
            pltpu.sync_copy(eb_ref.at[t], idxb)
            pltpu.sync_copy(ev_ref.at[t], idxv)
            cps = [
                pltpu.async_copy(base_ref.at[idxb.at[k]],
                                 rows.at[pl.ds(k * SUB, SUB)], sem)
                for k in range(KSUB)
            ]
            for cp in cps:
                cp.wait()

            def grp(g, c):
                for u in range(L):
                    r = g * L + u
                    prow = rows[r]
                    a = jnp.take_along_axis(prow, idxa_v, axis=0)
                    b = jnp.take_along_axis(prow, idxb_v, axis=0)
                    pay[r] = a * b
                return c

            lax.fori_loop(0, CH // L, grp, 0)
            for k in range(KSUB):
                pltpu.sync_copy(pay.at[pl.ds(k * SUB, SUB)],
                                acc.at[idxv.at[k]], add=True)
            return carry

        lax.fori_loop(0, NITER, it, 0)
        plsc.subcore_barrier()
        pltpu.sync_copy(acc.at[pl.ds(r0, RS)], out_ref.at[cid, pl.ds(r0, RS)])

    return sc1


def _make_sc_proj(E, V, Vp, RS):
    mesh = plsc.VectorSubcoreMesh(core_axis_name="c", subcore_axis_name="s")
    TCH = E // CH
    DUMMY = V

    @functools.partial(
        pl.kernel,
        out_type=jax.ShapeDtypeStruct((NC, Vp, L), jnp.float32),
        mesh=mesh,
        compiler_params=pltpu.CompilerParams(use_tc_tiling_on_sc=False),
        scratch_types=[
            pltpu.VMEM_SHARED((Vp, L), jnp.float32),
            pltpu.VMEM((len(_CTAB), L), jnp.int32),
            pltpu.VMEM((L,), jnp.int32),
            pltpu.VMEM((KSUB, SUB), jnp.int32),
            pltpu.VMEM((KSUB, SUB), jnp.int32),
            pltpu.VMEM((KSUB, SUB), jnp.int32),
            pltpu.VMEM((CH, L), jnp.float32),
            pltpu.VMEM((CH, L), jnp.float32),
            pltpu.VMEM((CH, L), jnp.float32),
            pltpu.VMEM((L,), jnp.int32),
            pltpu.VMEM((L,), jnp.float32),
            pltpu.SemaphoreType.DMA,
        ],
    )
    def sc3(base_ref, vox_ref, eb_ref, ev_ref, starts_ref, ctab_ref,
            zeros_ref, out_ref,
            acc, ctab, starts, idxb, idxv, sidx, rows, vrows, pay,
            prevbuf, rbuf, sem):
        cid = lax.axis_index("c")
        sid = lax.axis_index("s")
        gw = cid * NS + sid
        lane = _lane()
        r0 = sid * RS
        pltpu.sync_copy(ctab_ref, ctab)
        pltpu.sync_copy(starts_ref.at[gw], starts)
        pltpu.sync_copy(zeros_ref, acc.at[pl.ds(r0, RS)])
        idxp_v = ctab[2]
        idxm_v = ctab[3]
        sh3_v = ctab[4]
        sh6_v = ctab[5]
        packf_v = ctab[6]
        signf = ctab[7].astype(jnp.float32)
        shd1_v = ctab[8]
        spl15_v = ctab[9]
        ind0_v = ctab[10]
        shu1_v = ctab[11]
        ind15_f = ctab[12].astype(jnp.float32)
        zsplat = jnp.zeros((L,), jnp.int32)
        sev = starts[...]
        sv = sev[0]
        es = sev[1]
        t0 = sv // CH
        prevbuf[...] = jnp.full((L,), -1, jnp.int32)
        rbuf[...] = jnp.zeros((L,), jnp.float32)
        plsc.subcore_barrier()

        def chunk(i, carry):
            t = t0 + i

            @pl.when(t * CH < es)
            def _():
                pltpu.sync_copy(eb_ref.at[t], idxb)
                pltpu.sync_copy(ev_ref.at[t], idxv)
                cps = []
                for k in range(KSUB):
                    cps.append(pltpu.async_copy(
                        base_ref.at[idxb.at[k]],
                        rows.at[pl.ds(k * SUB, SUB)], sem))
                    cps.append(pltpu.async_copy(
                        vox_ref.at[idxv.at[k]],
                        vrows.at[pl.ds(k * SUB, SUB)], sem))
                for cp in cps:
                    cp.wait()

                def fin(gp, vvp, okp, samep, nextsame, mgf):
                    shifted = jnp.take_along_axis(samep, shu1_v, axis=0)
                    nxt = shifted * (1.0 - ind15_f) + nextsame * ind15_f
                    ki = ((1.0 - nxt) * okp.astype(jnp.float32) * mgf
                          ).astype(jnp.int32)
                    kkp = gp // KSUB
                    jjp = (gp % KSUB) * L
                    sidx[kkp, pl.ds(jjp, L)] = (vvp * ki
                                                + DUMMY * (1 - ki))

                def grp(g, st):
                    prevv, R, samep, vvp, okp = st
                    kk = g // KSUB
                    jj = (g % KSUB) * L
                    vv = idxv[kk, pl.ds(jj, L)]
                    sh = jnp.take_along_axis(vv, shd1_v, axis=0)
                    sh0 = prevv * ind0_v + sh * (1 - ind0_v)
                    same_f = (1 - jnp.minimum(jnp.abs(vv - sh0), 1)
                              ).astype(jnp.float32)
                    gi = jnp.full((L,), t * CH + g * L, jnp.int32) + lane
                    okv = (jnp.minimum(jnp.maximum(gi - sv + 1, 0), 1)
                           * jnp.minimum(jnp.maximum(es - gi, 0), 1))
                    firstg = jnp.take_along_axis(same_f, zsplat, axis=0)
                    mgf = jnp.minimum(g, 1).astype(jnp.float32)
                    fin(jnp.maximum(g - 1, 0), vvp, okp, samep, firstg, mgf)
                    for u in range(L):
                        r = g * L + u
                        prow = rows[r]
                        vrow = vrows[r]
                        dv = (jnp.take_along_axis(prow, idxp_v, axis=0)
                              - jnp.take_along_axis(vrow, idxm_v, axis=0))
                        q = dv * vrow
                        ss = (q + jnp.take_along_axis(q, sh3_v, axis=0)
                              + jnp.take_along_axis(q, sh6_v, axis=0))
                        f = jnp.take_along_axis(ss, packf_v, axis=0) * signf
                        su = jnp.take_along_axis(
                            same_f, jnp.full((L,), u, jnp.int32), axis=0)
                        R = su * jnp.maximum(R, f) + (1.0 - su) * f
                        pay[r] = R * signf
                    prev2 = jnp.take_along_axis(vv, spl15_v, axis=0)
                    return (prev2, R, same_f, vv, okv)

                st = lax.fori_loop(
                    0, CH // L, grp,
                    (prevbuf[...], rbuf[...], jnp.zeros((L,), jnp.float32),
                     jnp.full((L,), -1, jnp.int32), jnp.zeros((L,), jnp.int32)))
                prevbuf[...] = st[0]
                rbuf[...] = st[1]
                fin(CH // L - 1, st[3], st[4], st[2],
                    jnp.zeros((L,), jnp.float32), jnp.float32(1.0))
                for k in range(KSUB):
                    pltpu.sync_copy(pay.at[pl.ds(k * SUB, SUB)],
                                    acc.at[sidx.at[k]])

            return carry

        lax.fori_loop(0, TCH, chunk, 0)
        plsc.subcore_barrier()
        pltpu.sync_copy(acc.at[pl.ds(r0, RS)], out_ref.at[cid, pl.ds(r0, RS)])

    return sc3


def _tc_starts(ev2):
    Ew = ev2.shape[1]
    E = NW * Ew

    def body(ev_ref, out_ref):
        v2 = ev_ref[...]
        head = jnp.concatenate([v2[:1, :1] - 1, v2[:-1, -1:]], axis=0)
        prev2 = jnp.concatenate([head, v2[:, :-1]], axis=1)
        bound = v2 != prev2
        gidx = (lax.broadcasted_iota(jnp.int32, (NW, Ew), 0) * Ew
                + lax.broadcasted_iota(jnp.int32, (NW, Ew), 1))
        idxs = jnp.where(bound, gidx, E)
        bmin = jnp.min(idxs, axis=1)
        i32 = lax.iota(jnp.int32, NW)
        suf = jnp.min(
            jnp.where(i32[None, :] >= i32[:, None], bmin[None, :], E),
            axis=1)
        out_ref[...] = suf

    return pl.pallas_call(
        body,
        in_specs=[pl.BlockSpec(memory_space=pltpu.VMEM)],
        out_specs=pl.BlockSpec(memory_space=pltpu.VMEM),
        out_shape=jax.ShapeDtypeStruct((NW,), jnp.int32),
    )(ev2)


def _tc_stats(parts, bcenter):
    V = bcenter.shape[0]
    BR = 2048
    grid = (pl.cdiv(V, BR),)
    SWEEPS = 3
    ORDER = ((0, 2), (1, 2), (0, 1))

    def body(p_ref, c_ref, vol_ref, bxyz_ref, w_ref, v_ref, pack_ref):
        p = p_ref[0] + p_ref[1]
        n = p[:, 0]
        safe = jnp.maximum(n, 1.0)
        mean4 = p[:, 1:5] / safe[:, None]
        mask = n > 0.5
        bxyz = jnp.where(mask[:, None], mean4, c_ref[...])
        bxyz_ref[...] = bxyz
        vol_ref[...] = n
        mx, my, mz = mean4[:, 1], mean4[:, 2], mean4[:, 3]
        s2 = p[:, 5:11] / safe[:, None]

        a = {(0, 0): s2[:, 0] - mx * mx, (0, 1): s2[:, 1] - mx * my,
             (0, 2): s2[:, 2] - mx * mz, (1, 1): s2[:, 3] - my * my,
             (1, 2): s2[:, 4] - my * mz, (2, 2): s2[:, 5] - mz * mz}
        one = jnp.ones_like(a[(0, 0)])
        zero = jnp.zeros_like(one)
        vcols = [[one, zero, zero], [zero, one, zero], [zero, zero, one]]

        def ga(i, j):
            return a[(i, j)] if i <= j else a[(j, i)]

        def sa(i, j, val):
            a[(i, j) if i <= j else (j, i)] = val

        for _ in range(SWEEPS):
            for (pp, qq) in ORDER:
                rr = 3 - pp - qq
                apq = ga(pp, qq)
                app = ga(pp, pp)
                aqq = ga(qq, qq)
                tau = (aqq - app) / (2.0 * apq)
                t = jnp.sign(tau) / (jnp.abs(tau) + jnp.sqrt(1.0 + tau * tau))
                t = jnp.where(apq == 0.0, 0.0, t)
                c = 1.0 / jnp.sqrt(1.0 + t * t)
                s = t * c
                apr = ga(pp, rr)
                aqr = ga(qq, rr)
                sa(pp, pp, c * (c * app - s * apq) - s * (c * apq - s * aqq))
                sa(qq, qq, s * (s * app + c * apq) + c * (s * apq + c * aqq))
                sa(pp, qq, c * (s * app + c * apq) - s * (s * apq + c * aqq))
                sa(pp, rr, c * apr - s * aqr)
                sa(qq, rr, s * apr + c * aqr)
                for row in range(3):
                    vp = vcols[row][pp]
                    vq = vcols[row][qq]
                    vcols[row][pp] = c * vp - s * vq
                    vcols[row][qq] = s * vp + c * vq

        w = [ga(0, 0), ga(1, 1), ga(2, 2)]
        r0 = ((w[1] < w[0]).astype(jnp.int32)
              + (w[2] < w[0]).astype(jnp.int32))
        r1 = ((w[0] <= w[1]).astype(jnp.int32)
              + (w[2] < w[1]).astype(jnp.int32))
        r2 = ((w[0] <= w[2]).astype(jnp.int32)
              + (w[1] <= w[2]).astype(jnp.int32))
        ranks = [r0, r1, r2]

        def pick(vals, k):
            out = jnp.zeros_like(vals[0])
            for j in range(3):
                out = jnp.where(ranks[j] == k, vals[j], out)
            return out

        ws = [pick(w, k) for k in range(3)]
        w_ref[...] = jnp.stack(ws, axis=1)
        vs = [[pick(vcols[row], k) for k in range(3)] for row in range(3)]
        flat = [vs[row][k] for row in range(3) for k in range(3)]
        v_ref[...] = jnp.stack(flat, axis=1)
        pack_ref[...] = jnp.stack(
            flat + [bxyz[:, 1], bxyz[:, 2], bxyz[:, 3],
                    zero, zero, zero, zero], axis=1)

    return pl.pallas_call(
        body,
        grid=grid,
        in_specs=[
            pl.BlockSpec((2, BR, L), lambda i: (0, i, 0)),
            pl.BlockSpec((BR, 4), lambda i: (i, 0)),
        ],
        out_specs=[
            pl.BlockSpec((BR,), lambda i: (i,)),
            pl.BlockSpec((BR, 4), lambda i: (i, 0)),
            pl.BlockSpec((BR, 3), lambda i: (i, 0)),
            pl.BlockSpec((BR, 9), lambda i: (i, 0)),
            pl.BlockSpec((BR, L), lambda i: (i, 0)),
        ],
        out_shape=[
            jax.ShapeDtypeStruct((V,), jnp.float32),
            jax.ShapeDtypeStruct((V, 4), jnp.float32),
            jax.ShapeDtypeStruct((V, 3), jnp.float32),
            jax.ShapeDtypeStruct((V, 9), jnp.float32),
            jax.ShapeDtypeStruct((V, L), jnp.float32),
        ],
    )(parts, bcenter)


def _tc_combine(parts):
    Vp = parts.shape[1]
    BR = 2048
    grid = (pl.cdiv(Vp, BR),)

    def body(p_ref, mx_ref, mn_ref):
        p = p_ref[0] + p_ref[1]
        mx_ref[...] = p[:, 0:3]
        mn_ref[...] = p[:, 3:6]

    return pl.pallas_call(
        body,
        grid=grid,
        in_specs=[pl.BlockSpec((2, BR, L), lambda i: (0, i, 0))],
        out_specs=[
            pl.BlockSpec((BR, 3), lambda i: (i, 0)),
            pl.BlockSpec((BR, 3), lambda i: (i, 0)),
        ],
        out_shape=[
            jax.ShapeDtypeStruct((Vp, 3), jnp.float32),
            jax.ShapeDtypeStruct((Vp, 3), jnp.float32),
        ],
    )(parts)


def kernel(base_bxyz, bcenter, e_base, e_voxel):
    f32 = jnp.float32
    N = base_bxyz.shape[0]
    V = bcenter.shape[0]
    E = e_base.shape[0]
    Vp = ((V + 1 + 127) // 128) * 128
    RS = Vp // NS
    NITER = E // NW // CH

    base_pad = jnp.concatenate(
        [base_bxyz.astype(f32),
         jnp.ones((N, 1), f32),
         jnp.zeros((N, L - 5), f32)], axis=1)
    eb4 = e_base.reshape(E // CH, KSUB, SUB)
    ev4 = e_voxel.reshape(E // CH, KSUB, SUB)
    zeros = jnp.zeros((RS, L), f32)
    ctab = jnp.array(_CTAB, jnp.int32)

    starts = _tc_starts(e_voxel.reshape(NW, E // NW))
    st = jnp.concatenate([starts, jnp.full((1,), E, jnp.int32)])
    starts_full = jnp.concatenate(
        [st[:NW, None], st[1:NW + 1, None],
         jnp.zeros((NW, L - 2), jnp.int32)], axis=1)

    parts1 = _make_sc_moments(E, V, NITER, Vp, RS)(
        base_pad, eb4, ev4, ctab, zeros)
    vol, bxyz, eigvals, eigv9, voxpack = _tc_stats(parts1, bcenter)
    mask = vol > 0.5
    eigvecs = eigv9.reshape(V, 3, 3)
    parts3 = _make_sc_proj(E, V, Vp, RS)(
        base_pad, voxpack, eb4, ev4, starts_full, ctab, zeros)
    pmaxp, pminp = _tc_combine(parts3)
    return (bxyz, vol, mask, eigvals, eigvecs,
            pmaxp[:V], pminp[:V])
